# baseline (speedup 1.0000x reference)
import functools

import jax
import jax.numpy as jnp
from jax import lax
from jax.experimental import pallas as pl
from jax.experimental.pallas import tpu as pltpu
from jax.experimental.pallas import tpu_sc as plsc

F, C, H, W = 16, 128, 64, 64
N, M = 2048, 4096
NC, NS = 2, 16
NW = NC * NS
PW = N // NW
OWN = N // NS
LN = 16
E = 2 * M


def _balayer_body(feats_hbm, img_hbm, x_hbm, y_hbm, tracks_hbm,
                  assoc_out, samp_out,
                  srcf_v, tgtf_v, esrc_v, eoff_v, perm_v, islast_v,
                  lold_v, lown_v, ranks_v, aself_v, assoc_v,
                  myflag_v, flagbuf_v,
                  img_v, x_v, y_v, idx_v, w_v, rows_v, out_v,
                  labels_sp, flags_sp, sem):
    cid = lax.axis_index("c")
    sid = lax.axis_index("s")
    wid = sid * NC + cid
    base = wid * PW
    lane = lax.iota(jnp.int32, LN)
    zeros = jnp.zeros((LN,), jnp.int32)

    pltpu.sync_copy(img_hbm.at[pl.ds(base, PW)], img_v)
    pltpu.sync_copy(x_hbm.at[pl.ds(base, PW)], x_v)
    pltpu.sync_copy(y_hbm.at[pl.ds(base, PW)], y_v)

    for g in range(PW // LN):
        sl = pl.ds(g * LN, LN)
        xg = x_v[sl]
        yg = y_v[sl]
        im = img_v[sl]
        x0 = jnp.minimum(jnp.maximum(xg.astype(jnp.int32), 0), W - 2)
        y0 = jnp.minimum(jnp.maximum(yg.astype(jnp.int32), 0), H - 2)
        wx = xg - x0.astype(jnp.float32)
        wy = yg - y0.astype(jnp.float32)
        bidx = im * (H * W) + y0 * W + x0
        idx_v[0, sl] = bidx
        idx_v[1, sl] = bidx + 1
        idx_v[2, sl] = bidx + W
        idx_v[3, sl] = bidx + W + 1
        w_v[0, sl] = (1.0 - wy) * (1.0 - wx)
        w_v[1, sl] = (1.0 - wy) * wx
        w_v[2, sl] = wy * (1.0 - wx)
        w_v[3, sl] = wy * wx

    copies = [pltpu.async_copy(feats_hbm.at[idx_v.at[k]], rows_v.at[k], sem)
              for k in range(4)]

    @pl.when(cid == 0)
    def _association():
        nbase = sid * OWN

        pltpu.sync_copy(tracks_hbm.at[0], srcf_v.at[pl.ds(0, M)])
        pltpu.sync_copy(tracks_hbm.at[1], srcf_v.at[pl.ds(M, M)])
        pltpu.sync_copy(tracks_hbm.at[1], tgtf_v.at[pl.ds(0, M)])
        pltpu.sync_copy(tracks_hbm.at[0], tgtf_v.at[pl.ds(M, M)])

        def init_full(i, c):
            lold_v[pl.ds(i * LN, LN)] = i * LN + lane
            return c
        lax.fori_loop(0, N // LN, init_full, 0)

        def init_own(i, c):
            lown_v[pl.ds(i * LN, LN)] = nbase + i * LN + lane
            return c
        lax.fori_loop(0, OWN // LN, init_own, 0)

        def part_body(v, cnt):
            sl2 = pl.ds(v * LN, LN)
            s = srcf_v[sl2]
            t = tgtf_v[sl2]
            own = (t // OWN) == sid
            plsc.store_compressed(esrc_v.at[pl.ds(cnt, LN)], s, mask=own)
            plsc.store_compressed(eoff_v.at[pl.ds(cnt, LN)], t - nbase,
                                  mask=own)
            return cnt + jnp.max(plsc.all_reduce_population_count(own))
        cnt = lax.fori_loop(0, E // LN, part_body, jnp.int32(0))
        esrc_v[pl.ds(cnt, LN)] = zeros + nbase
        eoff_v[pl.ds(cnt, LN)] = zeros
        nv = (cnt + LN - 1) // LN

        def sort_body(v, c):
            sl2 = pl.ds(v * LN, LN)
            tk, pm = plsc.sort_key_val(eoff_v[sl2], lane)
            nxt = jnp.minimum(lane + 1, LN - 1)
            kn = jnp.take_along_axis(tk, nxt, axis=0)
            eoff_v[sl2] = tk
            perm_v[sl2] = pm
            islast_v[sl2] = jnp.where(
                jnp.logical_or(kn != tk, lane == LN - 1), 1, 0)
            return c
        lax.fori_loop(0, nv, sort_body, 0)

        def round_body(r, active):
            @pl.when(active)
            def _pass():
                myflag_v[...] = zeros

                def edge_body(v, c):
                    sl2 = pl.ds(v * LN, LN)
                    m = plsc.load_gather(lold_v, [esrc_v[sl2]])
                    tk = eoff_v[sl2]
                    mv = jnp.take_along_axis(m, perm_v[sl2], axis=0)
                    for d in (1, 2, 4, 8):
                        sh = jnp.maximum(lane - d, 0)
                        k2 = jnp.take_along_axis(tk, sh, axis=0)
                        m2 = jnp.take_along_axis(mv, sh, axis=0)
                        same = jnp.logical_and(k2 == tk, lane >= d)
                        mv = jnp.where(same, jnp.minimum(mv, m2), mv)
                    is_last = islast_v[sl2] != 0
                    cur = plsc.load_gather(lown_v, [tk])
                    upd = jnp.minimum(cur, mv)
                    need = jnp.logical_and(upd < cur, is_last)
                    plsc.store_scatter(lown_v, [tk], upd, mask=need)
                    myflag_v[...] = myflag_v[...] | jnp.where(need, 1, 0)
                    return c
                lax.fori_loop(0, nv, edge_body, 0)

                pltpu.sync_copy(lown_v, labels_sp.at[pl.ds(nbase, OWN)])
                pltpu.sync_copy(myflag_v, flags_sp.at[sid])

            plsc.subcore_barrier()

            @pl.when(active)
            def _refresh():
                pltpu.sync_copy(flags_sp, flagbuf_v)

            acc = zeros
            for i in range(NS):
                acc = acc | flagbuf_v[i]
            anyf = jnp.any(acc != 0)

            for i in range(NS):
                @pl.when(jnp.logical_and(active,
                                         jnp.any(flagbuf_v[i] != 0)))
                def _refresh_slice(i=i):
                    pltpu.sync_copy(labels_sp.at[pl.ds(i * OWN, OWN)],
                                    lold_v.at[pl.ds(i * OWN, OWN)])

            plsc.subcore_barrier()
            return jnp.logical_and(active, anyf)

        lax.fori_loop(0, F, round_body, jnp.bool_(True))

        @pl.when(sid == 0)
        def _finalize():
            def rank_body(i, carry):
                sl2 = pl.ds(i * LN, LN)
                v = lold_v[sl2]
                idxv = i * LN + lane
                selfm = v == idxv
                csum = plsc.cumsum(jnp.where(selfm, 1, 0).astype(jnp.int32))
                rk = carry + csum - 1
                ranks_v[sl2] = rk
                aself_v[sl2] = jnp.where(selfm, rk, 0)
                return carry + jnp.max(csum)
            lax.fori_loop(0, N // LN, rank_body, jnp.int32(0))

            def fin_body(i, c):
                sl2 = pl.ds(i * LN, LN)
                v = lold_v[sl2]
                idxv = i * LN + lane
                selfm = v == idxv
                gathered = plsc.load_gather(aself_v, [v])
                assoc_v[sl2] = jnp.where(selfm, ranks_v[sl2], gathered)
                return c
            lax.fori_loop(0, N // LN, fin_body, 0)

            pltpu.sync_copy(assoc_v, assoc_out)

    for cp in copies:
        cp.wait()

    ksplat = [jnp.full((LN,), k, jnp.int32) for k in range(4)]
    for g in range(PW // LN):
        sl = pl.ds(g * LN, LN)
        nid = g * LN + lane
        w0 = w_v[0, sl]
        w1 = w_v[1, sl]
        w2 = w_v[2, sl]
        w3 = w_v[3, sl]

        def chan_body(c, carry, nid=nid, w0=w0, w1=w1, w2=w2, w3=w3):
            cc = zeros + c
            acc = w0 * plsc.load_gather(rows_v, [ksplat[0], nid, cc])
            acc = acc + w1 * plsc.load_gather(rows_v, [ksplat[1], nid, cc])
            acc = acc + w2 * plsc.load_gather(rows_v, [ksplat[2], nid, cc])
            acc = acc + w3 * plsc.load_gather(rows_v, [ksplat[3], nid, cc])
            plsc.store_scatter(out_v, [nid, cc], acc)
            return carry
        lax.fori_loop(0, C, chan_body, 0)

    pltpu.sync_copy(out_v, samp_out.at[pl.ds(base, PW)])


@jax.jit
def _balayer_sc(feats_t, img, x, y, tracks):
    run = pl.kernel(
        _balayer_body,
        out_type=(jax.ShapeDtypeStruct((N,), jnp.int32),
                  jax.ShapeDtypeStruct((N, C), jnp.float32)),
        mesh=plsc.VectorSubcoreMesh(core_axis_name="c", subcore_axis_name="s"),
        compiler_params=pltpu.CompilerParams(needs_layout_passes=False),
        scratch_types=[
            pltpu.VMEM((E,), jnp.int32),
            pltpu.VMEM((E,), jnp.int32),
            pltpu.VMEM((E + LN,), jnp.int32),
            pltpu.VMEM((E + LN,), jnp.int32),
            pltpu.VMEM((E + LN,), jnp.int32),
            pltpu.VMEM((E + LN,), jnp.int32),
            pltpu.VMEM((N,), jnp.int32),
            pltpu.VMEM((OWN,), jnp.int32),
            pltpu.VMEM((N,), jnp.int32),
            pltpu.VMEM((N,), jnp.int32),
            pltpu.VMEM((N,), jnp.int32),
            pltpu.VMEM((LN,), jnp.int32),
            pltpu.VMEM((NS, LN), jnp.int32),
            pltpu.VMEM((PW,), jnp.int32),
            pltpu.VMEM((PW,), jnp.float32),
            pltpu.VMEM((PW,), jnp.float32),
            pltpu.VMEM((4, PW), jnp.int32),
            pltpu.VMEM((4, PW), jnp.float32),
            pltpu.VMEM((4, PW, C), jnp.float32),
            pltpu.VMEM((PW, C), jnp.float32),
            pltpu.VMEM_SHARED((N,), jnp.int32),
            pltpu.VMEM_SHARED((NS, LN), jnp.int32),
            pltpu.SemaphoreType.DMA,
        ],
    )
    return run(feats_t, img, x, y, tracks)


def kernel(proj_mats, feats, feat_img, feat_loc, tracks):
    del proj_mats
    feats_t = jnp.transpose(feats, (0, 2, 3, 1)).reshape(F * H * W, C)
    img = feat_img[:, 0]
    x = feat_loc[:, 0]
    y = feat_loc[:, 1]
    return _balayer_sc(feats_t, img, x, y, tracks)


# speedup vs baseline: 1.0462x; 1.0462x over previous
import functools

import jax
import jax.numpy as jnp
from jax import lax
from jax.experimental import pallas as pl
from jax.experimental.pallas import tpu as pltpu
from jax.experimental.pallas import tpu_sc as plsc

F, C, H, W = 16, 128, 64, 64
N, M = 2048, 4096
NC, NS = 2, 16
NW = NC * NS
PW = N // NW
OWN = N // NS
LN = 16
E = 2 * M


def _balayer_body(feats_hbm, img_hbm, x_hbm, y_hbm, tracks_hbm,
                  assoc_out, samp_out,
                  srcf_v, tgtf_v, esrc_v, eoff_v, perm_v, islast_v,
                  lold_v, lown_v, ranks_v, aself_v, assoc_v,
                  myflag_v, flagbuf_v, accflag_v,
                  img_v, x_v, y_v, idx_v, w_v, rows_v, out_v,
                  labels_sp, flags_sp, sem):
    cid = lax.axis_index("c")
    sid = lax.axis_index("s")
    wid = sid * NC + cid
    base = wid * PW
    lane = lax.iota(jnp.int32, LN)
    zeros = jnp.zeros((LN,), jnp.int32)

    pltpu.sync_copy(img_hbm.at[pl.ds(base, PW)], img_v)
    pltpu.sync_copy(x_hbm.at[pl.ds(base, PW)], x_v)
    pltpu.sync_copy(y_hbm.at[pl.ds(base, PW)], y_v)

    for g in range(PW // LN):
        sl = pl.ds(g * LN, LN)
        xg = x_v[sl]
        yg = y_v[sl]
        im = img_v[sl]
        x0 = jnp.minimum(jnp.maximum(xg.astype(jnp.int32), 0), W - 2)
        y0 = jnp.minimum(jnp.maximum(yg.astype(jnp.int32), 0), H - 2)
        wx = xg - x0.astype(jnp.float32)
        wy = yg - y0.astype(jnp.float32)
        bidx = im * (H * W) + y0 * W + x0
        idx_v[0, sl] = bidx
        idx_v[1, sl] = bidx + 1
        idx_v[2, sl] = bidx + W
        idx_v[3, sl] = bidx + W + 1
        w_v[0, sl] = (1.0 - wy) * (1.0 - wx)
        w_v[1, sl] = (1.0 - wy) * wx
        w_v[2, sl] = wy * (1.0 - wx)
        w_v[3, sl] = wy * wx

    copies = [pltpu.async_copy(feats_hbm.at[idx_v.at[k]], rows_v.at[k], sem)
              for k in range(4)]

    @pl.when(cid == 0)
    def _association():
        nbase = sid * OWN

        pltpu.sync_copy(tracks_hbm.at[0], srcf_v.at[pl.ds(0, M)])
        pltpu.sync_copy(tracks_hbm.at[1], srcf_v.at[pl.ds(M, M)])
        pltpu.sync_copy(tracks_hbm.at[1], tgtf_v.at[pl.ds(0, M)])
        pltpu.sync_copy(tracks_hbm.at[0], tgtf_v.at[pl.ds(M, M)])

        def init_full(i, c):
            lold_v[pl.ds(i * LN, LN)] = i * LN + lane
            return c
        lax.fori_loop(0, N // LN, init_full, 0)

        def init_own(i, c):
            lown_v[pl.ds(i * LN, LN)] = nbase + i * LN + lane
            return c
        lax.fori_loop(0, OWN // LN, init_own, 0)

        def part_body(v, cnt):
            sl2 = pl.ds(v * LN, LN)
            s = srcf_v[sl2]
            t = tgtf_v[sl2]
            own = (t // OWN) == sid
            plsc.store_compressed(esrc_v.at[pl.ds(cnt, LN)], s, mask=own)
            plsc.store_compressed(eoff_v.at[pl.ds(cnt, LN)], t - nbase,
                                  mask=own)
            return cnt + jnp.max(plsc.all_reduce_population_count(own))
        cnt = lax.fori_loop(0, E // LN, part_body, jnp.int32(0))
        esrc_v[pl.ds(cnt, LN)] = zeros + nbase
        eoff_v[pl.ds(cnt, LN)] = zeros
        nv = (cnt + LN - 1) // LN

        def sort_body(v, c):
            sl2 = pl.ds(v * LN, LN)
            tk, pm = plsc.sort_key_val(eoff_v[sl2], lane)
            nxt = jnp.minimum(lane + 1, LN - 1)
            kn = jnp.take_along_axis(tk, nxt, axis=0)
            eoff_v[sl2] = tk
            perm_v[sl2] = pm
            islast_v[sl2] = jnp.where(
                jnp.logical_or(kn != tk, lane == LN - 1), 1, 0)
            return c
        lax.fori_loop(0, nv, sort_body, 0)

        def round_body(r, active):
            p = r & 1

            @pl.when(active)
            def _pass():
                myflag_v[...] = zeros

                def edge_body(v, c):
                    sl2 = pl.ds(v * LN, LN)
                    m = plsc.load_gather(lold_v, [esrc_v[sl2]])
                    tk = eoff_v[sl2]
                    mv = jnp.take_along_axis(m, perm_v[sl2], axis=0)
                    for d in (1, 2, 4, 8):
                        sh = jnp.maximum(lane - d, 0)
                        k2 = jnp.take_along_axis(tk, sh, axis=0)
                        m2 = jnp.take_along_axis(mv, sh, axis=0)
                        same = jnp.logical_and(k2 == tk, lane >= d)
                        mv = jnp.where(same, jnp.minimum(mv, m2), mv)
                    is_last = islast_v[sl2] != 0
                    cur = plsc.load_gather(lown_v, [tk])
                    upd = jnp.minimum(cur, mv)
                    need = jnp.logical_and(upd < cur, is_last)
                    plsc.store_scatter(lown_v, [tk], upd, mask=need)
                    myflag_v[...] = myflag_v[...] | jnp.where(need, 1, 0)
                    return c
                lax.fori_loop(0, nv, edge_body, 0)

                pltpu.sync_copy(lown_v,
                                labels_sp.at[pl.ds(p * N + nbase, OWN)])
                pltpu.sync_copy(myflag_v,
                                flags_sp.at[pl.ds((p * NS + sid) * LN, LN)])

                plsc.subcore_barrier()

                pltpu.sync_copy(flags_sp.at[pl.ds(p * NS * LN, NS * LN)],
                                flagbuf_v)
                acc = zeros
                for i in range(NS):
                    acc = acc | flagbuf_v[pl.ds(i * LN, LN)]
                accflag_v[...] = acc

                for i in range(NS):
                    @pl.when(jnp.any(flagbuf_v[pl.ds(i * LN, LN)] != 0))
                    def _refresh_slice(i=i):
                        pltpu.sync_copy(
                            labels_sp.at[pl.ds(p * N + i * OWN, OWN)],
                            lold_v.at[pl.ds(i * OWN, OWN)])

            return jnp.logical_and(active, jnp.any(accflag_v[...] != 0))

        lax.fori_loop(0, F, round_body, jnp.bool_(True))

        @pl.when(sid == 0)
        def _finalize():
            def rank_body(i, carry):
                sl2 = pl.ds(i * LN, LN)
                v = lold_v[sl2]
                idxv = i * LN + lane
                selfm = v == idxv
                csum = plsc.cumsum(jnp.where(selfm, 1, 0).astype(jnp.int32))
                rk = carry + csum - 1
                ranks_v[sl2] = rk
                aself_v[sl2] = jnp.where(selfm, rk, 0)
                return carry + jnp.max(csum)
            lax.fori_loop(0, N // LN, rank_body, jnp.int32(0))

            def fin_body(i, c):
                sl2 = pl.ds(i * LN, LN)
                v = lold_v[sl2]
                idxv = i * LN + lane
                selfm = v == idxv
                gathered = plsc.load_gather(aself_v, [v])
                assoc_v[sl2] = jnp.where(selfm, ranks_v[sl2], gathered)
                return c
            lax.fori_loop(0, N // LN, fin_body, 0)

            pltpu.sync_copy(assoc_v, assoc_out)

    for cp in copies:
        cp.wait()

    ksplat = [jnp.full((LN,), k, jnp.int32) for k in range(4)]
    for g in range(PW // LN):
        sl = pl.ds(g * LN, LN)
        nid = g * LN + lane
        w0 = w_v[0, sl]
        w1 = w_v[1, sl]
        w2 = w_v[2, sl]
        w3 = w_v[3, sl]

        def chan_body(c, carry, nid=nid, w0=w0, w1=w1, w2=w2, w3=w3):
            cc = zeros + c
            acc = w0 * plsc.load_gather(rows_v, [ksplat[0], nid, cc])
            acc = acc + w1 * plsc.load_gather(rows_v, [ksplat[1], nid, cc])
            acc = acc + w2 * plsc.load_gather(rows_v, [ksplat[2], nid, cc])
            acc = acc + w3 * plsc.load_gather(rows_v, [ksplat[3], nid, cc])
            plsc.store_scatter(out_v, [nid, cc], acc)
            return carry
        lax.fori_loop(0, C, chan_body, 0)

    pltpu.sync_copy(out_v, samp_out.at[pl.ds(base, PW)])


@jax.jit
def _balayer_sc(feats_t, img, x, y, tracks):
    run = pl.kernel(
        _balayer_body,
        out_type=(jax.ShapeDtypeStruct((N,), jnp.int32),
                  jax.ShapeDtypeStruct((N, C), jnp.float32)),
        mesh=plsc.VectorSubcoreMesh(core_axis_name="c", subcore_axis_name="s"),
        compiler_params=pltpu.CompilerParams(needs_layout_passes=False),
        scratch_types=[
            pltpu.VMEM((E,), jnp.int32),
            pltpu.VMEM((E,), jnp.int32),
            pltpu.VMEM((E + LN,), jnp.int32),
            pltpu.VMEM((E + LN,), jnp.int32),
            pltpu.VMEM((E + LN,), jnp.int32),
            pltpu.VMEM((E + LN,), jnp.int32),
            pltpu.VMEM((N,), jnp.int32),
            pltpu.VMEM((OWN,), jnp.int32),
            pltpu.VMEM((N,), jnp.int32),
            pltpu.VMEM((N,), jnp.int32),
            pltpu.VMEM((N,), jnp.int32),
            pltpu.VMEM((LN,), jnp.int32),
            pltpu.VMEM((NS * LN,), jnp.int32),
            pltpu.VMEM((LN,), jnp.int32),
            pltpu.VMEM((PW,), jnp.int32),
            pltpu.VMEM((PW,), jnp.float32),
            pltpu.VMEM((PW,), jnp.float32),
            pltpu.VMEM((4, PW), jnp.int32),
            pltpu.VMEM((4, PW), jnp.float32),
            pltpu.VMEM((4, PW, C), jnp.float32),
            pltpu.VMEM((PW, C), jnp.float32),
            pltpu.VMEM_SHARED((2 * N,), jnp.int32),
            pltpu.VMEM_SHARED((2 * NS * LN,), jnp.int32),
            pltpu.SemaphoreType.DMA,
        ],
    )
    return run(feats_t, img, x, y, tracks)


def kernel(proj_mats, feats, feat_img, feat_loc, tracks):
    del proj_mats
    feats_t = jnp.transpose(feats, (0, 2, 3, 1)).reshape(F * H * W, C)
    img = feat_img[:, 0]
    x = feat_loc[:, 0]
    y = feat_loc[:, 1]
    return _balayer_sc(feats_t, img, x, y, tracks)


# speedup vs baseline: 1.2434x; 1.1886x over previous
import functools

import jax
import jax.numpy as jnp
from jax import lax
from jax.experimental import pallas as pl
from jax.experimental.pallas import tpu as pltpu
from jax.experimental.pallas import tpu_sc as plsc

F, C, H, W = 16, 128, 64, 64
N, M = 2048, 4096
NC, NS = 2, 16
NW = NC * NS
PW = N // NW
OWN = N // NS
LN = 16
E = 2 * M


def _balayer_body(feats_hbm, img_hbm, x_hbm, y_hbm, tracks_hbm,
                  assoc_out, samp_out,
                  srcf_v, tgtf_v, esrc_v, eoff_v, perm_v, islast_v,
                  lold_v, lown_v, ranks_v, aself_v, assoc_v,
                  myflag_v, flagbuf_v, accflag_v,
                  img_v, x_v, y_v, idx_v, w_v, rows_v, out_v,
                  labels_sp, flags_sp, sem, sem2):
    cid = lax.axis_index("c")
    sid = lax.axis_index("s")
    wid = sid * NC + cid
    base = wid * PW
    lane = lax.iota(jnp.int32, LN)
    zeros = jnp.zeros((LN,), jnp.int32)

    pltpu.sync_copy(img_hbm.at[pl.ds(base, PW)], img_v)
    pltpu.sync_copy(x_hbm.at[pl.ds(base, PW)], x_v)
    pltpu.sync_copy(y_hbm.at[pl.ds(base, PW)], y_v)

    for g in range(PW // LN):
        sl = pl.ds(g * LN, LN)
        xg = x_v[sl]
        yg = y_v[sl]
        im = img_v[sl]
        x0 = jnp.minimum(jnp.maximum(xg.astype(jnp.int32), 0), W - 2)
        y0 = jnp.minimum(jnp.maximum(yg.astype(jnp.int32), 0), H - 2)
        wx = xg - x0.astype(jnp.float32)
        wy = yg - y0.astype(jnp.float32)
        bidx = im * (H * W) + y0 * W + x0
        idx_v[0, sl] = bidx
        idx_v[1, sl] = bidx + 1
        idx_v[2, sl] = bidx + W
        idx_v[3, sl] = bidx + W + 1
        w_v[0, sl] = (1.0 - wy) * (1.0 - wx)
        w_v[1, sl] = (1.0 - wy) * wx
        w_v[2, sl] = wy * (1.0 - wx)
        w_v[3, sl] = wy * wx

    copies = [pltpu.async_copy(feats_hbm.at[idx_v.at[k]], rows_v.at[k], sem)
              for k in range(4)]

    @pl.when(cid == 0)
    def _association():
        nbase = sid * OWN

        pltpu.sync_copy(tracks_hbm.at[0], srcf_v.at[pl.ds(0, M)])
        pltpu.sync_copy(tracks_hbm.at[1], srcf_v.at[pl.ds(M, M)])
        pltpu.sync_copy(tracks_hbm.at[1], tgtf_v.at[pl.ds(0, M)])
        pltpu.sync_copy(tracks_hbm.at[0], tgtf_v.at[pl.ds(M, M)])

        def init_full(i, c):
            lold_v[pl.ds(i * LN, LN)] = i * LN + lane
            return c
        lax.fori_loop(0, N // LN, init_full, 0)

        def init_own(i, c):
            lown_v[pl.ds(i * LN, LN)] = nbase + i * LN + lane
            return c
        lax.fori_loop(0, OWN // LN, init_own, 0)

        def part_body(v, cnt):
            sl2 = pl.ds(v * LN, LN)
            s = srcf_v[sl2]
            t = tgtf_v[sl2]
            own = (t // OWN) == sid
            plsc.store_compressed(esrc_v.at[pl.ds(cnt, LN)], s, mask=own)
            plsc.store_compressed(eoff_v.at[pl.ds(cnt, LN)], t - nbase,
                                  mask=own)
            return cnt + jnp.max(plsc.all_reduce_population_count(own))
        cnt = lax.fori_loop(0, E // LN, part_body, jnp.int32(0))
        esrc_v[pl.ds(cnt, LN)] = zeros + nbase
        eoff_v[pl.ds(cnt, LN)] = zeros
        nv = (cnt + LN - 1) // LN

        def sort_body(v, c):
            sl2 = pl.ds(v * LN, LN)
            tk, pm = plsc.sort_key_val(eoff_v[sl2], lane)
            nxt = jnp.minimum(lane + 1, LN - 1)
            kn = jnp.take_along_axis(tk, nxt, axis=0)
            eoff_v[sl2] = tk
            perm_v[sl2] = pm
            islast_v[sl2] = jnp.where(
                jnp.logical_or(kn != tk, lane == LN - 1), 1, 0)
            return c
        lax.fori_loop(0, nv, sort_body, 0)

        def round_body(r, active):
            p = r & 1

            @pl.when(active)
            def _pass():
                myflag_v[...] = zeros

                def edge_body(v, c):
                    sl2 = pl.ds(v * LN, LN)
                    m = plsc.load_gather(lold_v, [esrc_v[sl2]])
                    tk = eoff_v[sl2]
                    mv = jnp.take_along_axis(m, perm_v[sl2], axis=0)
                    for d in (1, 2, 4, 8):
                        sh = jnp.maximum(lane - d, 0)
                        k2 = jnp.take_along_axis(tk, sh, axis=0)
                        m2 = jnp.take_along_axis(mv, sh, axis=0)
                        same = jnp.logical_and(k2 == tk, lane >= d)
                        mv = jnp.where(same, jnp.minimum(mv, m2), mv)
                    is_last = islast_v[sl2] != 0
                    cur = plsc.load_gather(lown_v, [tk])
                    upd = jnp.minimum(cur, mv)
                    need = jnp.logical_and(upd < cur, is_last)
                    plsc.store_scatter(lown_v, [tk], upd, mask=need)
                    myflag_v[...] = myflag_v[...] | jnp.where(need, 1, 0)
                    return c
                lax.fori_loop(0, nv, edge_body, 0)

                cpa = pltpu.async_copy(
                    lown_v, labels_sp.at[pl.ds(p * N + nbase, OWN)], sem2)
                cpb = pltpu.async_copy(
                    myflag_v, flags_sp.at[pl.ds((p * NS + sid) * LN, LN)],
                    sem2)
                cpa.wait()
                cpb.wait()

                plsc.subcore_barrier()

                cpc = pltpu.async_copy(
                    labels_sp.at[pl.ds(p * N, N)], lold_v, sem2)
                cpd = pltpu.async_copy(
                    flags_sp.at[pl.ds(p * NS * LN, NS * LN)], flagbuf_v,
                    sem2)
                cpc.wait()
                cpd.wait()
                acc = zeros
                for i in range(NS):
                    acc = acc | flagbuf_v[pl.ds(i * LN, LN)]
                accflag_v[...] = acc

            return jnp.logical_and(active, jnp.any(accflag_v[...] != 0))

        lax.fori_loop(0, F, round_body, jnp.bool_(True))

        @pl.when(sid == 0)
        def _finalize():
            def rank_body(i, carry):
                sl2 = pl.ds(i * LN, LN)
                v = lold_v[sl2]
                idxv = i * LN + lane
                selfm = v == idxv
                csum = plsc.cumsum(jnp.where(selfm, 1, 0).astype(jnp.int32))
                rk = carry + csum - 1
                ranks_v[sl2] = rk
                aself_v[sl2] = jnp.where(selfm, rk, 0)
                return carry + jnp.max(csum)
            lax.fori_loop(0, N // LN, rank_body, jnp.int32(0))

            def fin_body(i, c):
                sl2 = pl.ds(i * LN, LN)
                v = lold_v[sl2]
                idxv = i * LN + lane
                selfm = v == idxv
                gathered = plsc.load_gather(aself_v, [v])
                assoc_v[sl2] = jnp.where(selfm, ranks_v[sl2], gathered)
                return c
            lax.fori_loop(0, N // LN, fin_body, 0)

            pltpu.sync_copy(assoc_v, assoc_out)

    for cp in copies:
        cp.wait()

    ksplat = [jnp.full((LN,), k, jnp.int32) for k in range(4)]
    for g in range(PW // LN):
        sl = pl.ds(g * LN, LN)
        nid = g * LN + lane
        w0 = w_v[0, sl]
        w1 = w_v[1, sl]
        w2 = w_v[2, sl]
        w3 = w_v[3, sl]

        def chan_body(c, carry, nid=nid, w0=w0, w1=w1, w2=w2, w3=w3):
            cc = zeros + c
            acc = w0 * plsc.load_gather(rows_v, [ksplat[0], nid, cc])
            acc = acc + w1 * plsc.load_gather(rows_v, [ksplat[1], nid, cc])
            acc = acc + w2 * plsc.load_gather(rows_v, [ksplat[2], nid, cc])
            acc = acc + w3 * plsc.load_gather(rows_v, [ksplat[3], nid, cc])
            plsc.store_scatter(out_v, [nid, cc], acc)
            return carry
        lax.fori_loop(0, C, chan_body, 0)

    pltpu.sync_copy(out_v, samp_out.at[pl.ds(base, PW)])


@jax.jit
def _balayer_sc(feats_t, img, x, y, tracks):
    run = pl.kernel(
        _balayer_body,
        out_type=(jax.ShapeDtypeStruct((N,), jnp.int32),
                  jax.ShapeDtypeStruct((N, C), jnp.float32)),
        mesh=plsc.VectorSubcoreMesh(core_axis_name="c", subcore_axis_name="s"),
        compiler_params=pltpu.CompilerParams(needs_layout_passes=False),
        scratch_types=[
            pltpu.VMEM((E,), jnp.int32),
            pltpu.VMEM((E,), jnp.int32),
            pltpu.VMEM((E + LN,), jnp.int32),
            pltpu.VMEM((E + LN,), jnp.int32),
            pltpu.VMEM((E + LN,), jnp.int32),
            pltpu.VMEM((E + LN,), jnp.int32),
            pltpu.VMEM((N,), jnp.int32),
            pltpu.VMEM((OWN,), jnp.int32),
            pltpu.VMEM((N,), jnp.int32),
            pltpu.VMEM((N,), jnp.int32),
            pltpu.VMEM((N,), jnp.int32),
            pltpu.VMEM((LN,), jnp.int32),
            pltpu.VMEM((NS * LN,), jnp.int32),
            pltpu.VMEM((LN,), jnp.int32),
            pltpu.VMEM((PW,), jnp.int32),
            pltpu.VMEM((PW,), jnp.float32),
            pltpu.VMEM((PW,), jnp.float32),
            pltpu.VMEM((4, PW), jnp.int32),
            pltpu.VMEM((4, PW), jnp.float32),
            pltpu.VMEM((4, PW, C), jnp.float32),
            pltpu.VMEM((PW, C), jnp.float32),
            pltpu.VMEM_SHARED((2 * N,), jnp.int32),
            pltpu.VMEM_SHARED((2 * NS * LN,), jnp.int32),
            pltpu.SemaphoreType.DMA,
            pltpu.SemaphoreType.DMA,
        ],
    )
    return run(feats_t, img, x, y, tracks)


def kernel(proj_mats, feats, feat_img, feat_loc, tracks):
    del proj_mats
    feats_t = jnp.transpose(feats, (0, 2, 3, 1)).reshape(F * H * W, C)
    img = feat_img[:, 0]
    x = feat_loc[:, 0]
    y = feat_loc[:, 1]
    return _balayer_sc(feats_t, img, x, y, tracks)


# speedup vs baseline: 1.2709x; 1.0221x over previous
import functools

import jax
import jax.numpy as jnp
from jax import lax
from jax.experimental import pallas as pl
from jax.experimental.pallas import tpu as pltpu
from jax.experimental.pallas import tpu_sc as plsc

F, C, H, W = 16, 128, 64, 64
N, M = 2048, 4096
NC, NS = 2, 16
NW = NC * NS
PW = N // NW
OWN = N // NS
LN = 16
E = 2 * M


def _balayer_body(feats_hbm, img_hbm, x_hbm, y_hbm, tracks_hbm,
                  assoc_out, samp_out,
                  srcf_v, tgtf_v, esrc_v, eoff_v, perm_v, islast_v,
                  lold_v, lown_v, ranks_v, aself_v, assoc_v,
                  myflag_v, flagbuf_v, accflag_v,
                  img_v, x_v, y_v, idx_v, w_v, rows_v, out_v,
                  labels_sp, flags_sp, sem, sem2):
    cid = lax.axis_index("c")
    sid = lax.axis_index("s")
    wid = sid * NC + cid
    base = wid * PW
    lane = lax.iota(jnp.int32, LN)
    zeros = jnp.zeros((LN,), jnp.int32)

    stage = [pltpu.async_copy(img_hbm.at[pl.ds(base, PW)], img_v, sem2),
             pltpu.async_copy(x_hbm.at[pl.ds(base, PW)], x_v, sem2),
             pltpu.async_copy(y_hbm.at[pl.ds(base, PW)], y_v, sem2)]
    for cp in stage:
        cp.wait()

    for g in range(PW // LN):
        sl = pl.ds(g * LN, LN)
        xg = x_v[sl]
        yg = y_v[sl]
        im = img_v[sl]
        x0 = jnp.minimum(jnp.maximum(xg.astype(jnp.int32), 0), W - 2)
        y0 = jnp.minimum(jnp.maximum(yg.astype(jnp.int32), 0), H - 2)
        wx = xg - x0.astype(jnp.float32)
        wy = yg - y0.astype(jnp.float32)
        bidx = im * (H * W) + y0 * W + x0
        idx_v[0, sl] = bidx
        idx_v[1, sl] = bidx + 1
        idx_v[2, sl] = bidx + W
        idx_v[3, sl] = bidx + W + 1
        w_v[0, sl] = (1.0 - wy) * (1.0 - wx)
        w_v[1, sl] = (1.0 - wy) * wx
        w_v[2, sl] = wy * (1.0 - wx)
        w_v[3, sl] = wy * wx

    copies = [pltpu.async_copy(feats_hbm.at[idx_v.at[k]], rows_v.at[k], sem)
              for k in range(4)]

    @pl.when(cid == 0)
    def _association():
        nbase = sid * OWN

        ecp = [pltpu.async_copy(tracks_hbm.at[0], srcf_v.at[pl.ds(0, M)],
                                sem2),
               pltpu.async_copy(tracks_hbm.at[1], srcf_v.at[pl.ds(M, M)],
                                sem2),
               pltpu.async_copy(tracks_hbm.at[1], tgtf_v.at[pl.ds(0, M)],
                                sem2),
               pltpu.async_copy(tracks_hbm.at[0], tgtf_v.at[pl.ds(M, M)],
                                sem2)]
        for cp in ecp:
            cp.wait()

        def init_full(i, c):
            lold_v[pl.ds(i * LN, LN)] = i * LN + lane
            return c
        lax.fori_loop(0, N // LN, init_full, 0)

        def init_own(i, c):
            lown_v[pl.ds(i * LN, LN)] = nbase + i * LN + lane
            return c
        lax.fori_loop(0, OWN // LN, init_own, 0)

        def part_body(v, cnt):
            sl2 = pl.ds(v * LN, LN)
            s = srcf_v[sl2]
            t = tgtf_v[sl2]
            own = (t // OWN) == sid
            plsc.store_compressed(esrc_v.at[pl.ds(cnt, LN)], s, mask=own)
            plsc.store_compressed(eoff_v.at[pl.ds(cnt, LN)], t - nbase,
                                  mask=own)
            return cnt + jnp.max(plsc.all_reduce_population_count(own))
        cnt = lax.fori_loop(0, E // LN, part_body, jnp.int32(0))
        esrc_v[pl.ds(cnt, LN)] = zeros + nbase
        eoff_v[pl.ds(cnt, LN)] = zeros
        nv = (cnt + LN - 1) // LN

        def sort_body(v, c):
            sl2 = pl.ds(v * LN, LN)
            tk, pm = plsc.sort_key_val(eoff_v[sl2], lane)
            nxt = jnp.minimum(lane + 1, LN - 1)
            kn = jnp.take_along_axis(tk, nxt, axis=0)
            eoff_v[sl2] = tk
            perm_v[sl2] = pm
            islast_v[sl2] = jnp.where(
                jnp.logical_or(kn != tk, lane == LN - 1), 1, 0)
            return c
        lax.fori_loop(0, nv, sort_body, 0)

        def round_body(r, active):
            p = r & 1

            @pl.when(active)
            def _pass():
                myflag_v[...] = zeros

                def edge_body(v, c):
                    sl2 = pl.ds(v * LN, LN)
                    m = plsc.load_gather(lold_v, [esrc_v[sl2]])
                    tk = eoff_v[sl2]
                    mv = jnp.take_along_axis(m, perm_v[sl2], axis=0)
                    for d in (1, 2, 4, 8):
                        sh = jnp.maximum(lane - d, 0)
                        k2 = jnp.take_along_axis(tk, sh, axis=0)
                        m2 = jnp.take_along_axis(mv, sh, axis=0)
                        same = jnp.logical_and(k2 == tk, lane >= d)
                        mv = jnp.where(same, jnp.minimum(mv, m2), mv)
                    is_last = islast_v[sl2] != 0
                    cur = plsc.load_gather(lown_v, [tk])
                    upd = jnp.minimum(cur, mv)
                    need = jnp.logical_and(upd < cur, is_last)
                    plsc.store_scatter(lown_v, [tk], upd, mask=need)
                    myflag_v[...] = myflag_v[...] | jnp.where(need, 1, 0)
                    return c
                lax.fori_loop(0, nv, edge_body, 0)

                cpa = pltpu.async_copy(
                    lown_v, labels_sp.at[pl.ds(p * N + nbase, OWN)], sem2)
                cpb = pltpu.async_copy(
                    myflag_v, flags_sp.at[pl.ds((p * NS + sid) * LN, LN)],
                    sem2)
                cpa.wait()
                cpb.wait()

                plsc.subcore_barrier()

                cpc = pltpu.async_copy(
                    labels_sp.at[pl.ds(p * N, N)], lold_v, sem2)
                cpd = pltpu.async_copy(
                    flags_sp.at[pl.ds(p * NS * LN, NS * LN)], flagbuf_v,
                    sem2)
                cpc.wait()
                cpd.wait()
                acc = zeros
                for i in range(NS):
                    acc = acc | flagbuf_v[pl.ds(i * LN, LN)]
                accflag_v[...] = acc

            return jnp.logical_and(active, jnp.any(accflag_v[...] != 0))

        lax.fori_loop(0, F, round_body, jnp.bool_(True))

        @pl.when(sid == 0)
        def _finalize():
            def rank_body(i, carry):
                sl2 = pl.ds(i * LN, LN)
                v = lold_v[sl2]
                idxv = i * LN + lane
                selfm = v == idxv
                csum = plsc.cumsum(jnp.where(selfm, 1, 0).astype(jnp.int32))
                rk = carry + csum - 1
                ranks_v[sl2] = rk
                aself_v[sl2] = jnp.where(selfm, rk, 0)
                return carry + jnp.max(csum)
            lax.fori_loop(0, N // LN, rank_body, jnp.int32(0))

            def fin_body(i, c):
                sl2 = pl.ds(i * LN, LN)
                v = lold_v[sl2]
                idxv = i * LN + lane
                selfm = v == idxv
                gathered = plsc.load_gather(aself_v, [v])
                assoc_v[sl2] = jnp.where(selfm, ranks_v[sl2], gathered)
                return c
            lax.fori_loop(0, N // LN, fin_body, 0)

            pltpu.sync_copy(assoc_v, assoc_out)

    for cp in copies:
        cp.wait()

    ksplat = [jnp.full((LN,), k, jnp.int32) for k in range(4)]
    for g in range(PW // LN):
        sl = pl.ds(g * LN, LN)
        nid = g * LN + lane
        w0 = w_v[0, sl]
        w1 = w_v[1, sl]
        w2 = w_v[2, sl]
        w3 = w_v[3, sl]

        def chan_body(c, carry, nid=nid, w0=w0, w1=w1, w2=w2, w3=w3):
            cc = zeros + c
            acc = w0 * plsc.load_gather(rows_v, [ksplat[0], nid, cc])
            acc = acc + w1 * plsc.load_gather(rows_v, [ksplat[1], nid, cc])
            acc = acc + w2 * plsc.load_gather(rows_v, [ksplat[2], nid, cc])
            acc = acc + w3 * plsc.load_gather(rows_v, [ksplat[3], nid, cc])
            plsc.store_scatter(out_v, [nid, cc], acc)
            return carry
        lax.fori_loop(0, C, chan_body, 0)

    pltpu.sync_copy(out_v, samp_out.at[pl.ds(base, PW)])


@jax.jit
def _balayer_sc(feats_t, img, x, y, tracks):
    run = pl.kernel(
        _balayer_body,
        out_type=(jax.ShapeDtypeStruct((N,), jnp.int32),
                  jax.ShapeDtypeStruct((N, C), jnp.float32)),
        mesh=plsc.VectorSubcoreMesh(core_axis_name="c", subcore_axis_name="s"),
        compiler_params=pltpu.CompilerParams(needs_layout_passes=False),
        scratch_types=[
            pltpu.VMEM((E,), jnp.int32),
            pltpu.VMEM((E,), jnp.int32),
            pltpu.VMEM((E + LN,), jnp.int32),
            pltpu.VMEM((E + LN,), jnp.int32),
            pltpu.VMEM((E + LN,), jnp.int32),
            pltpu.VMEM((E + LN,), jnp.int32),
            pltpu.VMEM((N,), jnp.int32),
            pltpu.VMEM((OWN,), jnp.int32),
            pltpu.VMEM((N,), jnp.int32),
            pltpu.VMEM((N,), jnp.int32),
            pltpu.VMEM((N,), jnp.int32),
            pltpu.VMEM((LN,), jnp.int32),
            pltpu.VMEM((NS * LN,), jnp.int32),
            pltpu.VMEM((LN,), jnp.int32),
            pltpu.VMEM((PW,), jnp.int32),
            pltpu.VMEM((PW,), jnp.float32),
            pltpu.VMEM((PW,), jnp.float32),
            pltpu.VMEM((4, PW), jnp.int32),
            pltpu.VMEM((4, PW), jnp.float32),
            pltpu.VMEM((4, PW, C), jnp.float32),
            pltpu.VMEM((PW, C), jnp.float32),
            pltpu.VMEM_SHARED((2 * N,), jnp.int32),
            pltpu.VMEM_SHARED((2 * NS * LN,), jnp.int32),
            pltpu.SemaphoreType.DMA,
            pltpu.SemaphoreType.DMA,
        ],
    )
    return run(feats_t, img, x, y, tracks)


def kernel(proj_mats, feats, feat_img, feat_loc, tracks):
    del proj_mats
    feats_t = jnp.transpose(feats, (0, 2, 3, 1)).reshape(F * H * W, C)
    img = feat_img[:, 0]
    x = feat_loc[:, 0]
    y = feat_loc[:, 1]
    return _balayer_sc(feats_t, img, x, y, tracks)


# speedup vs baseline: 1.2724x; 1.0011x over previous
import jax
import jax.numpy as jnp
from jax import lax
from jax.experimental import pallas as pl
from jax.experimental.pallas import tpu as pltpu
from jax.experimental.pallas import tpu_sc as plsc

F, C, H, W = 16, 128, 64, 64
N, M = 2048, 4096
NC, NS = 2, 16
NW = NC * NS
PW = N // NW
OWN = N // NS
LN = 16
E = 2 * M


def _balayer_body(feats_hbm, img_hbm, x_hbm, y_hbm, tracks_hbm,
                  assoc_out, samp_out,
                  srcf_v, tgtf_v, esrc_v, eoff_v, perm_v, islast_v,
                  lold_v, lown_v, ranks_v, aself_v, assoc_v,
                  myflag_v, flagbuf_v, accflag_v,
                  img_v, x_v, y_v, idx_v, w_v, rows_v, out_v,
                  labels_sp, flags_sp, sem, sem2):
    cid = lax.axis_index("c")
    sid = lax.axis_index("s")
    wid = sid * NC + cid
    base = wid * PW
    lane = lax.iota(jnp.int32, LN)
    zeros = jnp.zeros((LN,), jnp.int32)

    stage = [pltpu.async_copy(img_hbm.at[pl.ds(base, PW)], img_v, sem2),
             pltpu.async_copy(x_hbm.at[pl.ds(base, PW)], x_v, sem2),
             pltpu.async_copy(y_hbm.at[pl.ds(base, PW)], y_v, sem2)]
    for cp in stage:
        cp.wait()

    for g in range(PW // LN):
        sl = pl.ds(g * LN, LN)
        xg = x_v[sl]
        yg = y_v[sl]
        im = img_v[sl]
        x0 = jnp.minimum(jnp.maximum(xg.astype(jnp.int32), 0), W - 2)
        y0 = jnp.minimum(jnp.maximum(yg.astype(jnp.int32), 0), H - 2)
        wx = xg - x0.astype(jnp.float32)
        wy = yg - y0.astype(jnp.float32)
        bidx = im * (H * W) + y0 * W + x0
        idx_v[0, sl] = bidx
        idx_v[1, sl] = bidx + 1
        idx_v[2, sl] = bidx + W
        idx_v[3, sl] = bidx + W + 1
        w_v[0, sl] = (1.0 - wy) * (1.0 - wx)
        w_v[1, sl] = (1.0 - wy) * wx
        w_v[2, sl] = wy * (1.0 - wx)
        w_v[3, sl] = wy * wx

    copies = [pltpu.async_copy(feats_hbm.at[idx_v.at[k]], rows_v.at[k], sem)
              for k in range(4)]

    @pl.when(cid == 0)
    def _association():
        nbase = sid * OWN

        ecp = [pltpu.async_copy(tracks_hbm.at[0], srcf_v.at[pl.ds(0, M)],
                                sem2),
               pltpu.async_copy(tracks_hbm.at[1], srcf_v.at[pl.ds(M, M)],
                                sem2),
               pltpu.async_copy(tracks_hbm.at[1], tgtf_v.at[pl.ds(0, M)],
                                sem2),
               pltpu.async_copy(tracks_hbm.at[0], tgtf_v.at[pl.ds(M, M)],
                                sem2)]
        for cp in ecp:
            cp.wait()

        def init_full(i, c):
            lold_v[pl.ds(i * LN, LN)] = i * LN + lane
            return c
        lax.fori_loop(0, N // LN, init_full, 0)

        def init_own(i, c):
            lown_v[pl.ds(i * LN, LN)] = nbase + i * LN + lane
            return c
        lax.fori_loop(0, OWN // LN, init_own, 0)

        def part_body(v, cnt):
            sl2 = pl.ds(v * LN, LN)
            s = srcf_v[sl2]
            t = tgtf_v[sl2]
            own = (t // OWN) == sid
            plsc.store_compressed(esrc_v.at[pl.ds(cnt, LN)], s, mask=own)
            plsc.store_compressed(eoff_v.at[pl.ds(cnt, LN)], t - nbase,
                                  mask=own)
            return cnt + jnp.max(plsc.all_reduce_population_count(own))
        cnt = lax.fori_loop(0, E // LN, part_body, jnp.int32(0))
        esrc_v[pl.ds(cnt, LN)] = zeros + nbase
        eoff_v[pl.ds(cnt, LN)] = zeros
        nv = (cnt + LN - 1) // LN

        def sort_body(v, c):
            sl2 = pl.ds(v * LN, LN)
            tk, pm = plsc.sort_key_val(eoff_v[sl2], lane)
            nxt = jnp.minimum(lane + 1, LN - 1)
            kn = jnp.take_along_axis(tk, nxt, axis=0)
            eoff_v[sl2] = tk
            perm_v[sl2] = pm
            islast_v[sl2] = jnp.where(
                jnp.logical_or(kn != tk, lane == LN - 1), 1, 0)
            return c
        lax.fori_loop(0, nv, sort_body, 0)

        def round_body(r, active):
            p = r & 1

            @pl.when(active)
            def _pass():
                myflag_v[...] = zeros

                def edge_body(v, c):
                    sl2 = pl.ds(v * LN, LN)
                    m = plsc.load_gather(lold_v, [esrc_v[sl2]])
                    tk = eoff_v[sl2]
                    mv = jnp.take_along_axis(m, perm_v[sl2], axis=0)
                    for d in (1, 2, 4, 8):
                        sh = jnp.maximum(lane - d, 0)
                        k2 = jnp.take_along_axis(tk, sh, axis=0)
                        m2 = jnp.take_along_axis(mv, sh, axis=0)
                        same = jnp.logical_and(k2 == tk, lane >= d)
                        mv = jnp.where(same, jnp.minimum(mv, m2), mv)
                    is_last = islast_v[sl2] != 0
                    cur = plsc.load_gather(lown_v, [tk])
                    upd = jnp.minimum(cur, mv)
                    need = jnp.logical_and(upd < cur, is_last)
                    plsc.store_scatter(lown_v, [tk], upd, mask=need)
                    myflag_v[...] = myflag_v[...] | jnp.where(need, 1, 0)
                    return c
                lax.fori_loop(0, nv, edge_body, 0)

                cpa = pltpu.async_copy(
                    lown_v, labels_sp.at[pl.ds(p * N + nbase, OWN)], sem2)
                cpb = pltpu.async_copy(
                    myflag_v, flags_sp.at[pl.ds((p * NS + sid) * LN, LN)],
                    sem2)
                cpa.wait()
                cpb.wait()

                plsc.subcore_barrier()

                cpc = pltpu.async_copy(
                    labels_sp.at[pl.ds(p * N, N)], lold_v, sem2)
                cpd = pltpu.async_copy(
                    flags_sp.at[pl.ds(p * NS * LN, NS * LN)], flagbuf_v,
                    sem2)
                cpc.wait()
                cpd.wait()
                acc = zeros
                for i in range(NS):
                    acc = acc | flagbuf_v[pl.ds(i * LN, LN)]
                accflag_v[...] = acc

            return jnp.logical_and(active, jnp.any(accflag_v[...] != 0))

        lax.fori_loop(0, F, round_body, jnp.bool_(True))

        @pl.when(sid == 0)
        def _finalize():
            def rank_body(i, carry):
                sl2 = pl.ds(i * LN, LN)
                v = lold_v[sl2]
                idxv = i * LN + lane
                selfm = v == idxv
                csum = plsc.cumsum(jnp.where(selfm, 1, 0).astype(jnp.int32))
                rk = carry + csum - 1
                ranks_v[sl2] = rk
                aself_v[sl2] = jnp.where(selfm, rk, 0)
                return carry + jnp.max(csum)
            lax.fori_loop(0, N // LN, rank_body, jnp.int32(0))

            def fin_body(i, c):
                sl2 = pl.ds(i * LN, LN)
                v = lold_v[sl2]
                idxv = i * LN + lane
                selfm = v == idxv
                gathered = plsc.load_gather(aself_v, [v])
                assoc_v[sl2] = jnp.where(selfm, ranks_v[sl2], gathered)
                return c
            lax.fori_loop(0, N // LN, fin_body, 0)

            pltpu.sync_copy(assoc_v, assoc_out)

    for cp in copies:
        cp.wait()

    ksplat = [jnp.full((LN,), k, jnp.int32) for k in range(4)]
    for g in range(PW // LN):
        sl = pl.ds(g * LN, LN)
        nid = g * LN + lane
        w0 = w_v[0, sl]
        w1 = w_v[1, sl]
        w2 = w_v[2, sl]
        w3 = w_v[3, sl]

        def chan_body(c, carry, nid=nid, w0=w0, w1=w1, w2=w2, w3=w3):
            cc = zeros + c
            acc = w0 * plsc.load_gather(rows_v, [ksplat[0], nid, cc])
            acc = acc + w1 * plsc.load_gather(rows_v, [ksplat[1], nid, cc])
            acc = acc + w2 * plsc.load_gather(rows_v, [ksplat[2], nid, cc])
            acc = acc + w3 * plsc.load_gather(rows_v, [ksplat[3], nid, cc])
            plsc.store_scatter(out_v, [nid, cc], acc)
            return carry
        lax.fori_loop(0, C, chan_body, 0)

    pltpu.sync_copy(out_v, samp_out.at[pl.ds(base, PW)])


@jax.jit
def _balayer_sc(feats_t, img, x, y, tracks):
    run = pl.kernel(
        _balayer_body,
        out_type=(jax.ShapeDtypeStruct((N,), jnp.int32),
                  jax.ShapeDtypeStruct((N, C), jnp.float32)),
        mesh=plsc.VectorSubcoreMesh(core_axis_name="c", subcore_axis_name="s"),
        compiler_params=pltpu.CompilerParams(needs_layout_passes=False),
        scratch_types=[
            pltpu.VMEM((E,), jnp.int32),
            pltpu.VMEM((E,), jnp.int32),
            pltpu.VMEM((E + LN,), jnp.int32),
            pltpu.VMEM((E + LN,), jnp.int32),
            pltpu.VMEM((E + LN,), jnp.int32),
            pltpu.VMEM((E + LN,), jnp.int32),
            pltpu.VMEM((N,), jnp.int32),
            pltpu.VMEM((OWN,), jnp.int32),
            pltpu.VMEM((N,), jnp.int32),
            pltpu.VMEM((N,), jnp.int32),
            pltpu.VMEM((N,), jnp.int32),
            pltpu.VMEM((LN,), jnp.int32),
            pltpu.VMEM((NS * LN,), jnp.int32),
            pltpu.VMEM((LN,), jnp.int32),
            pltpu.VMEM((PW,), jnp.int32),
            pltpu.VMEM((PW,), jnp.float32),
            pltpu.VMEM((PW,), jnp.float32),
            pltpu.VMEM((4, PW), jnp.int32),
            pltpu.VMEM((4, PW), jnp.float32),
            pltpu.VMEM((4, PW, C), jnp.float32),
            pltpu.VMEM((PW, C), jnp.float32),
            pltpu.VMEM_SHARED((2 * N,), jnp.int32),
            pltpu.VMEM_SHARED((2 * NS * LN,), jnp.int32),
            pltpu.SemaphoreType.DMA,
            pltpu.SemaphoreType.DMA,
        ],
    )
    return run(feats_t, img, x, y, tracks)


def kernel(proj_mats, feats, feat_img, feat_loc, tracks):
    del proj_mats
    feats_t = jnp.transpose(feats, (0, 2, 3, 1)).reshape(F * H * W, C)
    img = feat_img[:, 0]
    x = feat_loc[:, 0]
    y = feat_loc[:, 1]
    return _balayer_sc(feats_t, img, x, y, tracks)
